# Initial kernel scaffold; baseline (speedup 1.0000x reference)
import jax
import jax.numpy as jnp
from jax.experimental import pallas as pl


def kernel(x, weight, bias):
    raise NotImplementedError("write your pallas kernel here")


# speedup vs baseline: 5.2594x; 5.2594x over previous
import jax
import jax.numpy as jnp
from jax.experimental import pallas as pl
from jax.experimental.pallas import tpu as pltpu


def _linear_kernel(x_ref, w_ref, b_ref, o_ref):
    acc = jax.lax.dot_general(
        x_ref[...],
        w_ref[...],
        dimension_numbers=(((1,), (1,)), ((), ())),
        preferred_element_type=jnp.float32,
    )
    o_ref[...] = acc + b_ref[...]


def kernel(x, weight, bias):
    B, K = x.shape
    N = weight.shape[0]

    xb = x.astype(jnp.bfloat16)
    wb = weight.astype(jnp.bfloat16)
    b2 = bias.astype(jnp.float32).reshape(1, N)

    bm = min(B, 1024)
    bn = min(N, 1024)

    out = pl.pallas_call(
        _linear_kernel,
        out_shape=jax.ShapeDtypeStruct((B, N), jnp.float32),
        grid=(B // bm, N // bn),
        in_specs=[
            pl.BlockSpec((bm, K), lambda i, j: (i, 0)),
            pl.BlockSpec((bn, K), lambda i, j: (j, 0)),
            pl.BlockSpec((1, bn), lambda i, j: (0, j)),
        ],
        out_specs=pl.BlockSpec((bm, bn), lambda i, j: (i, j)),
        compiler_params=pltpu.CompilerParams(
            dimension_semantics=("parallel", "parallel"),
            vmem_limit_bytes=64 * 1024 * 1024,
        ),
        cost_estimate=pl.CostEstimate(
            flops=2 * B * N * K,
            transcendentals=0,
            bytes_accessed=2 * (B * K + N * K) + 4 * (B * N + N),
        ),
    )(xb, wb, b2)
    return out
